# baseline (speedup 1.0000x reference)
import functools

import jax
import jax.numpy as jnp
from jax import lax
from jax.experimental import pallas as pl
from jax.experimental.pallas import tpu as pltpu
from jax.experimental.pallas import tpu_sc as plsc

B, L = 4096, 200
D_ITEM, D_CAT, D_BRAND = 64, 32, 32
D_OUT = D_ITEM + D_CAT + D_BRAND
N = B * L
V_CAT = 1000

NC, NS = 2, 16
NW = NC * NS
TOK_PER_W = N // NW
C = 256
NPAIR = TOK_PER_W // (2 * C)
LANES = 16
CAT_ROWS = V_CAT * D_CAT // 128

_MESH = plsc.VectorSubcoreMesh(core_axis_name="c", subcore_axis_name="s")


@functools.partial(
    pl.kernel,
    out_type=jax.ShapeDtypeStruct((N, D_OUT), jnp.float32),
    mesh=_MESH,
    scratch_types=[
        pltpu.VMEM((C,), jnp.int32),
        pltpu.VMEM((C,), jnp.int32),
        pltpu.VMEM((C,), jnp.int32),
        pltpu.VMEM((C,), jnp.int32),
        pltpu.VMEM((C,), jnp.int32),
        pltpu.VMEM((C,), jnp.int32),
        pltpu.VMEM((C, D_OUT), jnp.float32),
        pltpu.VMEM((C, D_OUT), jnp.float32),
        pltpu.VMEM((CAT_ROWS, 128), jnp.float32),
        pltpu.SemaphoreType.DMA,
        pltpu.SemaphoreType.DMA,
        pltpu.SemaphoreType.DMA,
        pltpu.SemaphoreType.DMA,
        pltpu.SemaphoreType.DMA,
        pltpu.SemaphoreType.DMA,
        pltpu.SemaphoreType.DMA,
        pltpu.SemaphoreType.DMA,
    ],
    compiler_params=pltpu.CompilerParams(needs_layout_passes=False),
)
def _embed_sc(item_r, cat_r, brand_r, ti_r, tcflat_r, tb_r, out_r,
              idx_ia, idx_ca, idx_ba, idx_ib, idx_cb, idx_bb,
              out_va, out_vb, cat_v,
              sem_ia, sem_ib, sem_ga, sem_gb, sem_aa, sem_ab, sem_wa, sem_wb):
    wid = lax.axis_index("s") * NC + lax.axis_index("c")
    w_base = wid * TOK_PER_W

    pltpu.sync_copy(tcflat_r, cat_v)

    iota16 = lax.iota(jnp.int32, LANES)

    def fill_cat(idx_c, out_v):
        def group(k, carry):
            tvec = k * LANES + iota16
            rvec = idx_c[pl.ds(k * LANES, LANES)]
            base_flat = rvec * D_CAT
            for c in range(D_CAT):
                flat = base_flat + c
                row = lax.shift_right_logical(flat, 7)
                col = jnp.bitwise_and(flat, 127)
                vals = plsc.load_gather(cat_v, [row, col])
                plsc.store_scatter(out_v, [tvec, jnp.full((LANES,), D_ITEM + c, jnp.int32)], vals)
            return carry

        lax.fori_loop(0, C // LANES, group, 0)

    def do_pair(base_a, wait_writes):
        base_b = base_a + C
        if wait_writes:
            pltpu.make_async_copy(out_va, out_r.at[pl.ds(base_a, C)], sem_wa).wait()
            pltpu.make_async_copy(out_vb, out_r.at[pl.ds(base_b, C)], sem_wb).wait()

        i1 = pltpu.async_copy(item_r.at[pl.ds(base_a, C)], idx_ia, sem_ia)
        i2 = pltpu.async_copy(cat_r.at[pl.ds(base_a, C)], idx_ca, sem_ia)
        i3 = pltpu.async_copy(brand_r.at[pl.ds(base_a, C)], idx_ba, sem_ia)
        i4 = pltpu.async_copy(item_r.at[pl.ds(base_b, C)], idx_ib, sem_ib)
        i5 = pltpu.async_copy(cat_r.at[pl.ds(base_b, C)], idx_cb, sem_ib)
        i6 = pltpu.async_copy(brand_r.at[pl.ds(base_b, C)], idx_bb, sem_ib)

        i1.wait()
        i2.wait()
        i3.wait()
        ga = pltpu.async_copy(ti_r.at[idx_ia], out_va, sem_ga)
        i4.wait()
        i5.wait()
        i6.wait()
        gb = pltpu.async_copy(ti_r.at[idx_ib], out_vb, sem_gb)

        ga.wait()
        fill_cat(idx_ca, out_va)
        aa = pltpu.async_copy(tb_r.at[idx_ba], out_va, sem_aa, add=True)
        gb.wait()
        fill_cat(idx_cb, out_vb)
        ab = pltpu.async_copy(tb_r.at[idx_bb], out_vb, sem_ab, add=True)

        aa.wait()
        pltpu.async_copy(out_va, out_r.at[pl.ds(base_a, C)], sem_wa)
        ab.wait()
        pltpu.async_copy(out_vb, out_r.at[pl.ds(base_b, C)], sem_wb)

    do_pair(w_base, wait_writes=False)

    def pair(g, carry):
        do_pair(w_base + (2 * g) * C, wait_writes=True)
        return carry

    lax.fori_loop(1, NPAIR, pair, 0)

    last = w_base + (2 * NPAIR - 2) * C
    pltpu.make_async_copy(out_va, out_r.at[pl.ds(last, C)], sem_wa).wait()
    pltpu.make_async_copy(out_vb, out_r.at[pl.ds(last + C, C)], sem_wb).wait()


def kernel(item, category, brand, T_item, T_category, T_brand):
    item_f = item.reshape(N).astype(jnp.int32)
    cat_f = category.reshape(N).astype(jnp.int32)
    brand_f = brand.reshape(N).astype(jnp.int32)
    ti_p = jnp.pad(T_item, ((0, 0), (0, D_OUT - D_ITEM)))
    tc_flat = T_category.reshape(CAT_ROWS, 128)
    tb_p = jnp.pad(T_brand, ((0, 0), (D_OUT - D_BRAND, 0)))
    out = _embed_sc(item_f, cat_f, brand_f, ti_p, tc_flat, tb_p)
    return out.reshape(B, L, D_OUT)


# speedup vs baseline: 1.8217x; 1.8217x over previous
import functools

import jax
import jax.numpy as jnp
from jax import lax
from jax.experimental import pallas as pl
from jax.experimental.pallas import tpu as pltpu
from jax.experimental.pallas import tpu_sc as plsc

B, L = 4096, 200
D_ITEM, D_CAT, D_BRAND = 64, 32, 32
D_OUT = D_ITEM + D_CAT + D_BRAND
N = B * L

NC, NS = 2, 16
NW = NC * NS
TOK_PER_W = N // NW
C = 128
NSLOT = 4
NGROUP = TOK_PER_W // (NSLOT * C)

_MESH = plsc.VectorSubcoreMesh(core_axis_name="c", subcore_axis_name="s")

_SCRATCH = (
    [pltpu.VMEM((C,), jnp.int32) for _ in range(3 * NSLOT)]
    + [pltpu.VMEM((C, D_OUT), jnp.float32) for _ in range(NSLOT)]
    + [pltpu.SemaphoreType.DMA for _ in range(4 * NSLOT)]
)


@functools.partial(
    pl.kernel,
    out_type=jax.ShapeDtypeStruct((N, D_OUT), jnp.float32),
    mesh=_MESH,
    scratch_types=_SCRATCH,
)
def _embed_sc(item_r, cat_r, brand_r, ti_r, tc_r, tb_r, out_r, *scratch):
    idx_i = scratch[0:NSLOT]
    idx_c = scratch[NSLOT:2 * NSLOT]
    idx_b = scratch[2 * NSLOT:3 * NSLOT]
    out_v = scratch[3 * NSLOT:4 * NSLOT]
    sem_i = scratch[4 * NSLOT:5 * NSLOT]
    sem_g = scratch[5 * NSLOT:6 * NSLOT]
    sem_a = scratch[6 * NSLOT:7 * NSLOT]
    sem_w = scratch[7 * NSLOT:8 * NSLOT]

    wid = lax.axis_index("s") * NC + lax.axis_index("c")
    w_base = wid * TOK_PER_W

    def do_group(base, wait_writes):
        bases = [base + s * C for s in range(NSLOT)]
        if wait_writes:
            for s in range(NSLOT):
                pltpu.make_async_copy(out_v[s], out_r.at[pl.ds(bases[s], C)], sem_w[s]).wait()

        iw = []
        for s in range(NSLOT):
            i1 = pltpu.async_copy(item_r.at[pl.ds(bases[s], C)], idx_i[s], sem_i[s])
            i2 = pltpu.async_copy(cat_r.at[pl.ds(bases[s], C)], idx_c[s], sem_i[s])
            i3 = pltpu.async_copy(brand_r.at[pl.ds(bases[s], C)], idx_b[s], sem_i[s])
            iw.append((i1, i2, i3))

        gw = []
        for s in range(NSLOT):
            for d in iw[s]:
                d.wait()
            gw.append(pltpu.async_copy(ti_r.at[idx_i[s]], out_v[s], sem_g[s]))

        aw = []
        for s in range(NSLOT):
            gw[s].wait()
            a1 = pltpu.async_copy(tc_r.at[idx_c[s]], out_v[s], sem_a[s], add=True)
            a2 = pltpu.async_copy(tb_r.at[idx_b[s]], out_v[s], sem_a[s], add=True)
            aw.append((a1, a2))

        for s in range(NSLOT):
            aw[s][0].wait()
            aw[s][1].wait()
            pltpu.async_copy(out_v[s], out_r.at[pl.ds(bases[s], C)], sem_w[s])

    do_group(w_base, wait_writes=False)

    def group(g, carry):
        do_group(w_base + g * NSLOT * C, wait_writes=True)
        return carry

    lax.fori_loop(1, NGROUP, group, 0)

    last = w_base + (NGROUP - 1) * NSLOT * C
    for s in range(NSLOT):
        pltpu.make_async_copy(out_v[s], out_r.at[pl.ds(last + s * C, C)], sem_w[s]).wait()


def kernel(item, category, brand, T_item, T_category, T_brand):
    item_f = item.reshape(N).astype(jnp.int32)
    cat_f = category.reshape(N).astype(jnp.int32)
    brand_f = brand.reshape(N).astype(jnp.int32)
    ti_p = jnp.pad(T_item, ((0, 0), (0, D_OUT - D_ITEM)))
    tc_p = jnp.pad(T_category, ((0, 0), (D_ITEM, D_OUT - D_ITEM - D_CAT)))
    tb_p = jnp.pad(T_brand, ((0, 0), (D_OUT - D_BRAND, 0)))
    out = _embed_sc(item_f, cat_f, brand_f, ti_p, tc_p, tb_p)
    return out.reshape(B, L, D_OUT)


# speedup vs baseline: 1.8649x; 1.0237x over previous
import functools

import jax
import jax.numpy as jnp
from jax import lax
from jax.experimental import pallas as pl
from jax.experimental.pallas import tpu as pltpu
from jax.experimental.pallas import tpu_sc as plsc

B, L = 4096, 200
D_ITEM, D_CAT, D_BRAND = 64, 32, 32
D_OUT = D_ITEM + D_CAT + D_BRAND
N = B * L

NC, NS = 2, 16
NW = NC * NS
TOK_PER_W = N // NW
C = 128
NSLOT = 5
NGROUP = TOK_PER_W // (NSLOT * C)

_MESH = plsc.VectorSubcoreMesh(core_axis_name="c", subcore_axis_name="s")

_SCRATCH = (
    [pltpu.VMEM((C,), jnp.int32) for _ in range(3 * NSLOT)]
    + [pltpu.VMEM((C, D_OUT), jnp.float32) for _ in range(NSLOT)]
    + [pltpu.SemaphoreType.DMA for _ in range(4 * NSLOT)]
)


@functools.partial(
    pl.kernel,
    out_type=jax.ShapeDtypeStruct((N, D_OUT), jnp.float32),
    mesh=_MESH,
    scratch_types=_SCRATCH,
)
def _embed_sc(item_r, cat_r, brand_r, ti_r, tc_r, tb_r, out_r, *scratch):
    idx_i = scratch[0:NSLOT]
    idx_c = scratch[NSLOT:2 * NSLOT]
    idx_b = scratch[2 * NSLOT:3 * NSLOT]
    out_v = scratch[3 * NSLOT:4 * NSLOT]
    sem_i = scratch[4 * NSLOT:5 * NSLOT]
    sem_g = scratch[5 * NSLOT:6 * NSLOT]
    sem_a = scratch[6 * NSLOT:7 * NSLOT]
    sem_w = scratch[7 * NSLOT:8 * NSLOT]

    wid = lax.axis_index("s") * NC + lax.axis_index("c")
    w_base = wid * TOK_PER_W

    def do_group(base, wait_writes):
        bases = [base + s * C for s in range(NSLOT)]
        if wait_writes:
            for s in range(NSLOT):
                pltpu.make_async_copy(out_v[s], out_r.at[pl.ds(bases[s], C)], sem_w[s]).wait()

        iw = []
        for s in range(NSLOT):
            i1 = pltpu.async_copy(item_r.at[pl.ds(bases[s], C)], idx_i[s], sem_i[s])
            i2 = pltpu.async_copy(cat_r.at[pl.ds(bases[s], C)], idx_c[s], sem_i[s])
            i3 = pltpu.async_copy(brand_r.at[pl.ds(bases[s], C)], idx_b[s], sem_i[s])
            iw.append((i1, i2, i3))

        gw = []
        for s in range(NSLOT):
            for d in iw[s]:
                d.wait()
            gw.append(pltpu.async_copy(ti_r.at[idx_i[s]], out_v[s], sem_g[s]))

        aw = []
        for s in range(NSLOT):
            gw[s].wait()
            a1 = pltpu.async_copy(tc_r.at[idx_c[s]], out_v[s], sem_a[s], add=True)
            a2 = pltpu.async_copy(tb_r.at[idx_b[s]], out_v[s], sem_a[s], add=True)
            aw.append((a1, a2))

        for s in range(NSLOT):
            aw[s][0].wait()
            aw[s][1].wait()
            pltpu.async_copy(out_v[s], out_r.at[pl.ds(bases[s], C)], sem_w[s])

    do_group(w_base, wait_writes=False)

    def group(g, carry):
        do_group(w_base + g * NSLOT * C, wait_writes=True)
        return carry

    lax.fori_loop(1, NGROUP, group, 0)

    last = w_base + (NGROUP - 1) * NSLOT * C
    for s in range(NSLOT):
        pltpu.make_async_copy(out_v[s], out_r.at[pl.ds(last + s * C, C)], sem_w[s]).wait()


def kernel(item, category, brand, T_item, T_category, T_brand):
    item_f = item.reshape(N).astype(jnp.int32)
    cat_f = category.reshape(N).astype(jnp.int32)
    brand_f = brand.reshape(N).astype(jnp.int32)
    ti_p = jnp.pad(T_item, ((0, 0), (0, D_OUT - D_ITEM)))
    tc_p = jnp.pad(T_category, ((0, 0), (D_ITEM, D_OUT - D_ITEM - D_CAT)))
    tb_p = jnp.pad(T_brand, ((0, 0), (D_OUT - D_BRAND, 0)))
    out = _embed_sc(item_f, cat_f, brand_f, ti_p, tc_p, tb_p)
    return out.reshape(B, L, D_OUT)
